# baseline (speedup 1.0000x reference)
import jax
import jax.numpy as jnp
from jax import lax
from jax.experimental import pallas as pl
from jax.experimental.pallas import tpu as pltpu
from jax.experimental.pallas import tpu_sc as plsc

D = 64
B = 4096
NNEG = 128
L = 16
NC, NS = 2, 16
NW = NC * NS
BW = B // NW
DCH = D // L
JG = NNEG // L


def _pair_dot16(buf, row0, qv_of, lsb, lanes):
  out_lo = jnp.zeros((L,), jnp.float32)
  out_hi = jnp.zeros((L,), jnp.float32)
  for t in range(L):
    r = row0 + t
    qv = qv_of(t)
    p_lo = qv[0] * buf[r, pl.ds(0, L)]
    p_hi = qv[0] * buf[r, pl.ds(D, L)]
    for c in range(1, DCH):
      p_lo = p_lo + qv[c] * buf[r, pl.ds(c * L, L)]
      p_hi = p_hi + qv[c] * buf[r, pl.ds(D + c * L, L)]
    out_lo = jnp.where(lanes == t, jnp.sum(p_lo), out_lo)
    out_hi = jnp.where(lanes == t, jnp.sum(p_hi), out_hi)
  return jnp.where(lsb, out_hi, out_lo)


def _neg_compute(i, nbuf, qrow, nidx, nout, lanes):
  qv = [qrow[i, pl.ds(c * L, L)] for c in range(DCH)]
  for jg in range(JG):
    lsb = (nidx[i, pl.ds(jg * L, L)] & 1) == 1
    nout[i, pl.ds(jg * L, L)] = _pair_dot16(
        nbuf, jg * L, lambda t: qv, lsb, lanes)


def _body(et_hbm, rt_hbm, head_hbm, rel_hbm, ans_hbm, neg_hbm,
          scores_hbm, negsc_hbm,
          hidx, ridx, aidx, nidx, hpi, rpi, api, pnidx,
          hbuf, nega, negb, qrow, scr, nout, semg, sema, semb):
  wid = lax.axis_index("s") * NC + lax.axis_index("c")
  base = wid * BW
  lanes = lax.iota(jnp.int32, L)

  pltpu.sync_copy(head_hbm.at[pl.ds(base, BW)], hidx)
  pltpu.sync_copy(rel_hbm.at[pl.ds(base, BW)], ridx)
  pltpu.sync_copy(ans_hbm.at[pl.ds(base, BW)], aidx)
  pltpu.sync_copy(neg_hbm.at[pl.ds(base, BW)], nidx)

  for c in range(BW // L):
    sl = pl.ds(c * L, L)
    hpi[sl] = hidx[sl] >> 1
    rpi[sl] = ridx[sl] >> 1
    api[sl] = aidx[sl] >> 1

  def sbody(i, c):
    for ch in range(JG):
      sl = pl.ds(ch * L, L)
      pnidx[i, sl] = nidx[i, sl] >> 1
    return c
  lax.fori_loop(0, BW, sbody, 0)

  cph = pltpu.async_copy(et_hbm.at[hpi], hbuf, semg)
  cpr = pltpu.async_copy(rt_hbm.at[rpi], negb, semg)
  cph.wait()
  cpr.wait()

  def qbody(g, c):
    hl = hidx[pl.ds(g * L, L)] & 1
    rl = ridx[pl.ds(g * L, L)] & 1
    for t in range(L):
      i = g * L + t
      hs = hl[t]
      rs = rl[t]
      for ch in range(DCH):
        lo = pl.ds(ch * L, L)
        hi = pl.ds(D + ch * L, L)
        hv = jnp.where(hs == 1, hbuf[i, hi], hbuf[i, lo])
        rv = jnp.where(rs == 1, negb[i, hi], negb[i, lo])
        qrow[i, pl.ds(ch * L, L)] = hv * rv
    return c
  lax.fori_loop(0, BW // L, qbody, 0)

  pltpu.async_copy(et_hbm.at[api], nega, semg).wait()

  def pbody(g, c):
    lsb = (aidx[pl.ds(g * L, L)] & 1) == 1

    def qv_of(t):
      i = g * L + t
      return [qrow[i, pl.ds(cc * L, L)] for cc in range(DCH)]

    scr[pl.ds(pl.multiple_of(g * L, L), L)] = _pair_dot16(
        nega, g * L, qv_of, lsb, lanes)
    return c
  lax.fori_loop(0, BW // L, pbody, 0)

  pltpu.async_copy(et_hbm.at[pnidx.at[0]], nega, sema)

  def nbody(i2, c):
    qa = 2 * i2
    qb = qa + 1
    cpb = pltpu.async_copy(et_hbm.at[pnidx.at[qb]], negb, semb)
    pltpu.make_async_copy(et_hbm.at[pnidx.at[qa]], nega, sema).wait()
    _neg_compute(qa, nega, qrow, nidx, nout, lanes)

    @pl.when(i2 < BW // 2 - 1)
    def _():
      pltpu.async_copy(et_hbm.at[pnidx.at[qa + 2]], nega, sema)

    cpb.wait()
    _neg_compute(qb, negb, qrow, nidx, nout, lanes)
    return c
  lax.fori_loop(0, BW // 2, nbody, 0)

  pltpu.sync_copy(scr, scores_hbm.at[pl.ds(base, BW)])
  pltpu.sync_copy(nout, negsc_hbm.at[pl.ds(base, BW)])


@jax.jit
def kernel(entity_embedding, relation_embedding, head_idx, rel_idx, answer_idx, neg_idx):
  et = jnp.reshape(entity_embedding, (entity_embedding.shape[0] // 2, 2 * D))
  rt = jnp.reshape(relation_embedding, (relation_embedding.shape[0] // 2, 2 * D))
  mesh = plsc.VectorSubcoreMesh(core_axis_name="c", subcore_axis_name="s")
  run = pl.kernel(
      _body,
      out_type=(
          jax.ShapeDtypeStruct((B,), jnp.float32),
          jax.ShapeDtypeStruct((B, NNEG), jnp.float32),
      ),
      mesh=mesh,
      compiler_params=pltpu.CompilerParams(
          needs_layout_passes=False, use_tc_tiling_on_sc=True),
      scratch_types=[
          pltpu.VMEM((BW,), jnp.int32),
          pltpu.VMEM((BW,), jnp.int32),
          pltpu.VMEM((BW,), jnp.int32),
          pltpu.VMEM((BW, NNEG), jnp.int32),
          pltpu.VMEM((BW,), jnp.int32),
          pltpu.VMEM((BW,), jnp.int32),
          pltpu.VMEM((BW,), jnp.int32),
          pltpu.VMEM((BW, NNEG), jnp.int32),
          pltpu.VMEM((BW, 2 * D), jnp.float32),
          pltpu.VMEM((NNEG, 2 * D), jnp.float32),
          pltpu.VMEM((NNEG, 2 * D), jnp.float32),
          pltpu.VMEM((BW, D), jnp.float32),
          pltpu.VMEM((BW,), jnp.float32),
          pltpu.VMEM((BW, NNEG), jnp.float32),
          pltpu.SemaphoreType.DMA,
          pltpu.SemaphoreType.DMA,
          pltpu.SemaphoreType.DMA,
      ],
  )
  return run(et, rt, head_idx, rel_idx, answer_idx, neg_idx)
